# baseline (speedup 1.0000x reference)
import functools

import jax
import jax.numpy as jnp
from jax.experimental import pallas as pl

QB = 512
KB = 2048


def _dist_kernel(q_ref, k_ref, qs_ref, ks_ref, o_ref, *, n_keys):
    q = q_ref[...]
    kk = k_ref[...]
    acc = jax.lax.dot_general(
        q, kk, (((1,), (1,)), ((), ())),
        preferred_element_type=jnp.float32,
    )
    q_sq = qs_ref[...]
    k_sq = ks_ref[...]
    j = pl.program_id(1)
    col = j * KB + jax.lax.broadcasted_iota(jnp.int32, (1, KB), 1)
    d = q_sq - 2.0 * acc + k_sq
    d = jnp.where(col < n_keys, d, jnp.float32(jnp.inf))
    o_ref[...] = d


def kernel(queries, keys, k):
    Q, D = queries.shape
    K = keys.shape[0]
    KP = ((K + KB - 1) // KB) * KB
    keys_p = jnp.pad(keys, ((0, KP - K), (0, 0)))
    q_sq = jnp.sum(queries * queries, axis=1, keepdims=True)
    k_sq = jnp.pad(jnp.sum(keys * keys, axis=1), (0, KP - K))[None, :]
    dist = pl.pallas_call(
        functools.partial(_dist_kernel, n_keys=K),
        grid=(Q // QB, KP // KB),
        in_specs=[
            pl.BlockSpec((QB, D), lambda i, j: (i, 0)),
            pl.BlockSpec((KB, D), lambda i, j: (j, 0)),
            pl.BlockSpec((QB, 1), lambda i, j: (i, 0)),
            pl.BlockSpec((1, KB), lambda i, j: (0, j)),
        ],
        out_specs=pl.BlockSpec((QB, KB), lambda i, j: (i, j)),
        out_shape=jax.ShapeDtypeStruct((Q, KP), jnp.float32),
    )(queries, keys_p, q_sq, k_sq)
    neg, idx = jax.lax.top_k(-dist, D)
    return (-neg, idx.astype(jnp.int64))
